# baseline (speedup 1.0000x reference)
import functools

import jax
import jax.numpy as jnp
from jax import lax
from jax.experimental import pallas as pl
from jax.experimental.pallas import tpu as pltpu
from jax.experimental.pallas import tpu_sc as plsc

_N = 8388608
_F = 2 * _N
_NW = 32
_FPW = _F // _NW
_CF = 16384
_NCHUNK = _FPW // _CF
_L = 16
_BLK = 256


def _make_sc_kernel():
    mesh = plsc.VectorSubcoreMesh(core_axis_name="c", subcore_axis_name="s")

    @functools.partial(
        pl.kernel,
        mesh=mesh,
        out_type=jax.ShapeDtypeStruct((_F,), jnp.float32),
        scratch_types=[
            pltpu.VMEM((_CF,), jnp.float32),
            pltpu.VMEM((_CF,), jnp.float32),
            pltpu.VMEM((_CF,), jnp.float32),
            pltpu.VMEM((_CF,), jnp.float32),
            pltpu.SemaphoreType.DMA,
            pltpu.SemaphoreType.DMA,
            pltpu.SemaphoreType.DMA,
            pltpu.SemaphoreType.DMA,
        ],
        compiler_params=pltpu.CompilerParams(needs_layout_passes=False),
    )
    def _k(x_hbm, o_hbm, xb0, xb1, ob0, ob1, is0, is1, os0, os1):
        cid = lax.axis_index("c")
        sid = lax.axis_index("s")
        wid = sid * 2 + cid
        base = wid * _FPW
        fzero = jnp.zeros((_L,), jnp.float32)
        xbufs = (xb0, xb1)
        obufs = (ob0, ob1)
        isems = (is0, is1)
        osems = (os0, os1)

        def in_copy(ci, b):
            return pltpu.make_async_copy(
                x_hbm.at[pl.ds(base + ci * _CF, _CF)], xbufs[b], isems[b])

        def out_copy(ci, b):
            return pltpu.make_async_copy(
                obufs[b], o_hbm.at[pl.ds(base + ci * _CF, _CF)], osems[b])

        def compute(b):
            xbuf = xbufs[b]
            obuf = obufs[b]

            def blk_body(bi, c2):
                lbase = bi * _BLK
                for v in range(128 // _L):
                    lpos = lbase + v * _L
                    upos = lpos + 128
                    l = xbuf[pl.ds(lpos, _L)]
                    u = xbuf[pl.ds(upos, _L)]
                    keep_l = (l >= fzero) | (u > -l)
                    out_l = jnp.where(keep_l, l, fzero)
                    out_u = jnp.where(l >= fzero, u, jnp.maximum(u, fzero))
                    obuf[pl.ds(lpos, _L)] = out_l
                    obuf[pl.ds(upos, _L)] = out_u
                return c2

            lax.fori_loop(0, _CF // _BLK, blk_body, 0)

        in_copy(0, 0).start()

        @pl.loop(0, _NCHUNK, step=2)
        def _ring(g):
            for b in range(2):
                ci = g + b

                @pl.when(ci + 1 < _NCHUNK)
                def _start_next_in():
                    in_copy(ci + 1, 1 - b).start()

                in_copy(ci, b).wait()

                @pl.when(ci >= 2)
                def _wait_prev_out():
                    out_copy(ci - 2, b).wait()

                compute(b)
                out_copy(ci, b).start()

        out_copy(_NCHUNK - 2, 0).wait()
        out_copy(_NCHUNK - 1, 1).wait()

    return _k


_sc_kernel = _make_sc_kernel()


def kernel(bounds):
    n = bounds.shape[0]
    phys = bounds.reshape(n // 128, 128, 2).transpose(0, 2, 1).reshape(_F)
    out_phys = _sc_kernel(phys)
    return out_phys.reshape(n // 128, 2, 128).transpose(0, 2, 1).reshape(n, 2)
